# baseline (speedup 1.0000x reference)
import functools

import jax
import jax.numpy as jnp
from jax import lax
from jax.experimental import pallas as pl
from jax.experimental.pallas import tpu as pltpu
from jax.experimental.pallas import tpu_sc as plsc

N = 10000
E = 320000
D = 128
NC, NS = 2, 16
NW = NC * NS
EPW = E // NW
CH = 50
NCHUNK = EPW // CH
NBUF = 4
ROWS_PT = 624
TAIL0 = ROWS_PT * NS
TAILN = N - TAIL0

_sc_mesh = plsc.VectorSubcoreMesh(core_axis_name="c", subcore_axis_name="s")


def _deg_body(dst_hbm, hist_hbm, idx_v, hist_v, isem):
    c = lax.axis_index("c")
    s = lax.axis_index("s")
    wid = c * NS + s
    cp = pltpu.async_copy(dst_hbm.at[wid], idx_v, isem)

    zeros16 = jnp.zeros((16,), jnp.float32)

    @pl.loop(0, N // 16)
    def _zero(i):
        hist_v[pl.ds(i * 16, 16)] = zeros16

    cp.wait()

    ones16 = jnp.ones((16,), jnp.float32)

    @pl.loop(0, EPW // 16)
    def _acc(i):
        idx = idx_v[0, pl.ds(i * 16, 16)]
        plsc.addupdate_scatter(hist_v, [idx], ones16)

    pltpu.sync_copy(hist_v, hist_hbm.at[wid])


_deg_kernel = pl.kernel(
    _deg_body,
    out_type=jax.ShapeDtypeStruct((NW, N), jnp.float32),
    mesh=_sc_mesh,
    compiler_params=pltpu.CompilerParams(needs_layout_passes=False),
    scratch_types=[
        pltpu.VMEM((1, EPW), jnp.int32),
        pltpu.VMEM((N,), jnp.float32),
        pltpu.SemaphoreType.DMA,
    ],
)


def _gs_body(yw_hbm, edge_hbm, zero_hbm, parts_hbm,
             acc, idx_v, buf0, buf1, buf2, buf3,
             gsem0, gsem1, gsem2, gsem3):
    c = lax.axis_index("c")
    s = lax.axis_index("s")
    wid = c * NS + s
    src_v = idx_v.at[0]
    dst_v = idx_v.at[1]
    pltpu.sync_copy(edge_hbm.at[0, wid], src_v)

    bufs = (buf0, buf1, buf2, buf3)
    gsems = (gsem0, gsem1, gsem2, gsem3)

    for b in range(NBUF):
        pltpu.async_copy(yw_hbm.at[src_v.at[b]], bufs[b], gsems[b])

    pltpu.sync_copy(edge_hbm.at[1, wid], dst_v)

    row0 = s * ROWS_PT

    @pl.when(c == 0)
    def _():
        pltpu.sync_copy(yw_hbm.at[pl.ds(row0, ROWS_PT)],
                        acc.at[pl.ds(row0, ROWS_PT)])

        @pl.when(s == NS - 1)
        def _():
            pltpu.sync_copy(yw_hbm.at[pl.ds(TAIL0, TAILN)],
                            acc.at[pl.ds(TAIL0, TAILN)])

    @pl.when(c == 1)
    def _():
        pltpu.sync_copy(zero_hbm.at[pl.ds(row0, ROWS_PT)],
                        acc.at[pl.ds(row0, ROWS_PT)])

        @pl.when(s == NS - 1)
        def _():
            pltpu.sync_copy(zero_hbm.at[pl.ds(TAIL0, TAILN)],
                            acc.at[pl.ds(TAIL0, TAILN)])

    plsc.subcore_barrier()

    @pl.loop(0, NCHUNK, step=NBUF)
    def _main(j):
        for b in range(NBUF):
            k = j + b
            pltpu.make_async_copy(yw_hbm.at[src_v.at[k]],
                                  bufs[b], gsems[b]).wait()
            pltpu.sync_copy(bufs[b], acc.at[dst_v.at[k]], add=True)

            @pl.when(k + NBUF < NCHUNK)
            def _():
                pltpu.async_copy(yw_hbm.at[src_v.at[k + NBUF]],
                                 bufs[b], gsems[b])

    plsc.subcore_barrier()
    pltpu.sync_copy(acc.at[pl.ds(row0, ROWS_PT)],
                    parts_hbm.at[c, pl.ds(row0, ROWS_PT)])

    @pl.when(s == NS - 1)
    def _():
        pltpu.sync_copy(acc.at[pl.ds(TAIL0, TAILN)],
                        parts_hbm.at[c, pl.ds(TAIL0, TAILN)])


_gs_kernel = pl.kernel(
    _gs_body,
    out_type=jax.ShapeDtypeStruct((NC, N, D), jnp.float32),
    mesh=_sc_mesh,
    compiler_params=pltpu.CompilerParams(use_tc_tiling_on_sc=False),
    scratch_types=[
        pltpu.VMEM_SHARED((N, D), jnp.float32),
        pltpu.VMEM((2, NCHUNK, CH), jnp.int32),
        pltpu.VMEM((CH, D), jnp.float32),
        pltpu.VMEM((CH, D), jnp.float32),
        pltpu.VMEM((CH, D), jnp.float32),
        pltpu.VMEM((CH, D), jnp.float32),
        pltpu.SemaphoreType.DMA,
        pltpu.SemaphoreType.DMA,
        pltpu.SemaphoreType.DMA,
        pltpu.SemaphoreType.DMA,
    ],
)


_RB = 1000


def _dis_body(hist_ref, dis_ref):
    deg = 1.0 + jnp.sum(hist_ref[...], axis=0)
    dis_ref[...] = lax.rsqrt(deg)[:, None]


def _dis_kernel(hist):
    return pl.pallas_call(
        _dis_body,
        out_shape=jax.ShapeDtypeStruct((N, 1), jnp.float32),
    )(hist)


def _mm0_body(x_ref, dis_ref, w_ref, out_ref):
    xw = jnp.dot(x_ref[...], w_ref[...], preferred_element_type=jnp.float32)
    out_ref[...] = dis_ref[...] * xw


def _mm0_kernel(x, dis, W):
    grid = (N // _RB,)
    return pl.pallas_call(
        _mm0_body,
        grid=grid,
        in_specs=[
            pl.BlockSpec((_RB, D), lambda i: (i, 0)),
            pl.BlockSpec((_RB, 1), lambda i: (i, 0)),
            pl.BlockSpec((D, D), lambda i: (0, 0)),
        ],
        out_specs=pl.BlockSpec((_RB, D), lambda i: (i, 0)),
        out_shape=jax.ShapeDtypeStruct((N, D), jnp.float32),
    )(x, dis, W)


def _mid_body(p_ref, dis_ref, b_ref, w_ref, out_ref):
    dis = dis_ref[...]
    t = dis * (p_ref[0] + p_ref[1]) + b_ref[...]
    t = jnp.maximum(t, 0.0)
    tw = jnp.dot(t, w_ref[...], preferred_element_type=jnp.float32)
    out_ref[...] = dis * tw


def _mid_kernel(parts, dis, b, W):
    grid = (N // _RB,)
    return pl.pallas_call(
        _mid_body,
        grid=grid,
        in_specs=[
            pl.BlockSpec((NC, _RB, D), lambda i: (0, i, 0)),
            pl.BlockSpec((_RB, 1), lambda i: (i, 0)),
            pl.BlockSpec((1, D), lambda i: (0, 0)),
            pl.BlockSpec((D, D), lambda i: (0, 0)),
        ],
        out_specs=pl.BlockSpec((_RB, D), lambda i: (i, 0)),
        out_shape=jax.ShapeDtypeStruct((N, D), jnp.float32),
    )(parts, dis, b, W)


def _final_body(p_ref, dis_ref, b_ref, out_ref):
    out_ref[...] = dis_ref[...] * (p_ref[0] + p_ref[1]) + b_ref[...]


def _final_kernel(parts, dis, b):
    grid = (N // _RB,)
    return pl.pallas_call(
        _final_body,
        grid=grid,
        in_specs=[
            pl.BlockSpec((NC, _RB, D), lambda i: (0, i, 0)),
            pl.BlockSpec((_RB, 1), lambda i: (i, 0)),
            pl.BlockSpec((1, D), lambda i: (0, 0)),
        ],
        out_specs=pl.BlockSpec((_RB, D), lambda i: (i, 0)),
        out_shape=jax.ShapeDtypeStruct((N, D), jnp.float32),
    )(parts, dis, b)


def kernel(x, edge_index, W0, b0, W1, b1, W2, b2, W3, b3, W4, b4):
    edge_r = edge_index.reshape(2, NW, NCHUNK, CH)
    zeros = jnp.zeros((N, D), jnp.float32)

    hist = _deg_kernel(edge_index[1].reshape(NW, 1, EPW))
    dis = _dis_kernel(hist)

    bs = [b0.reshape(1, D), b1.reshape(1, D), b2.reshape(1, D),
          b3.reshape(1, D), b4.reshape(1, D)]
    Ws = [W0, W1, W2, W3, W4]

    yw = _mm0_kernel(x, dis, Ws[0])
    for l in range(4):
        parts = _gs_kernel(yw, edge_r, zeros)
        yw = _mid_kernel(parts, dis, bs[l], Ws[l + 1])
    parts = _gs_kernel(yw, edge_r, zeros)
    return _final_kernel(parts, dis, bs[4])


# speedup vs baseline: 1.0653x; 1.0653x over previous
import functools

import jax
import jax.numpy as jnp
from jax import lax
from jax.experimental import pallas as pl
from jax.experimental.pallas import tpu as pltpu
from jax.experimental.pallas import tpu_sc as plsc

N = 10000
E = 320000
D = 128
NC, NS = 2, 16
NW = NC * NS
EPW = E // NW
CH = 40
NCHUNK = EPW // CH
NBUF = 5
ROWS_PT = 624
TAIL0 = ROWS_PT * NS
TAILN = N - TAIL0

_sc_mesh = plsc.VectorSubcoreMesh(core_axis_name="c", subcore_axis_name="s")


def _deg_body(edge_hbm, hist_hbm, idx_v, hist_v, isem):
    c = lax.axis_index("c")
    s = lax.axis_index("s")
    wid = c * NS + s
    cp = pltpu.async_copy(edge_hbm.at[1, wid], idx_v, isem)

    zeros16 = jnp.zeros((16,), jnp.float32)

    @pl.loop(0, N // 16)
    def _zero(i):
        hist_v[pl.ds(i * 16, 16)] = zeros16

    cp.wait()

    ones16 = jnp.ones((16,), jnp.float32)

    @pl.loop(0, EPW // 16)
    def _acc(i):
        idx = idx_v[0, pl.ds(i * 16, 16)]
        plsc.addupdate_scatter(hist_v, [idx], ones16)

    pltpu.sync_copy(hist_v, hist_hbm.at[wid])


_deg_kernel = pl.kernel(
    _deg_body,
    out_type=jax.ShapeDtypeStruct((NW, N), jnp.float32),
    mesh=_sc_mesh,
    compiler_params=pltpu.CompilerParams(needs_layout_passes=False),
    scratch_types=[
        pltpu.VMEM((1, EPW), jnp.int32),
        pltpu.VMEM((N,), jnp.float32),
        pltpu.SemaphoreType.DMA,
    ],
)


def _gs_body(yw_hbm, edge_hbm, zero_hbm, parts_hbm,
             acc, idx_v, buf0, buf1, buf2, buf3, buf4,
             gsem0, gsem1, gsem2, gsem3, gsem4):
    c = lax.axis_index("c")
    s = lax.axis_index("s")
    wid = c * NS + s
    src_v = idx_v.at[0]
    dst_v = idx_v.at[1]
    pltpu.sync_copy(edge_hbm.at[0, wid], src_v)

    bufs = (buf0, buf1, buf2, buf3, buf4)
    gsems = (gsem0, gsem1, gsem2, gsem3, gsem4)

    for b in range(NBUF):
        pltpu.async_copy(yw_hbm.at[src_v.at[0, pl.ds(b * CH, CH)]],
                         bufs[b], gsems[b])

    pltpu.sync_copy(edge_hbm.at[1, wid], dst_v)

    row0 = s * ROWS_PT

    @pl.when(c == 0)
    def _():
        pltpu.sync_copy(yw_hbm.at[pl.ds(row0, ROWS_PT)],
                        acc.at[pl.ds(row0, ROWS_PT)])

        @pl.when(s == NS - 1)
        def _():
            pltpu.sync_copy(yw_hbm.at[pl.ds(TAIL0, TAILN)],
                            acc.at[pl.ds(TAIL0, TAILN)])

    @pl.when(c == 1)
    def _():
        pltpu.sync_copy(zero_hbm.at[pl.ds(row0, ROWS_PT)],
                        acc.at[pl.ds(row0, ROWS_PT)])

        @pl.when(s == NS - 1)
        def _():
            pltpu.sync_copy(zero_hbm.at[pl.ds(TAIL0, TAILN)],
                            acc.at[pl.ds(TAIL0, TAILN)])

    plsc.subcore_barrier()

    @pl.loop(0, NCHUNK, step=NBUF)
    def _main(j):
        for b in range(NBUF):
            k = j + b
            pltpu.make_async_copy(
                yw_hbm.at[src_v.at[0, pl.ds(k * CH, CH)]],
                bufs[b], gsems[b]).wait()
            pltpu.sync_copy(bufs[b],
                            acc.at[dst_v.at[0, pl.ds(k * CH, CH)]],
                            add=True)

            @pl.when(k + NBUF < NCHUNK)
            def _():
                pltpu.async_copy(
                    yw_hbm.at[src_v.at[0, pl.ds((k + NBUF) * CH, CH)]],
                    bufs[b], gsems[b])

    plsc.subcore_barrier()
    pltpu.sync_copy(acc.at[pl.ds(row0, ROWS_PT)],
                    parts_hbm.at[c, pl.ds(row0, ROWS_PT)])

    @pl.when(s == NS - 1)
    def _():
        pltpu.sync_copy(acc.at[pl.ds(TAIL0, TAILN)],
                        parts_hbm.at[c, pl.ds(TAIL0, TAILN)])


_gs_kernel = pl.kernel(
    _gs_body,
    out_type=jax.ShapeDtypeStruct((NC, N, D), jnp.float32),
    mesh=_sc_mesh,
    compiler_params=pltpu.CompilerParams(use_tc_tiling_on_sc=False),
    scratch_types=[
        pltpu.VMEM_SHARED((N, D), jnp.float32),
        pltpu.VMEM((2, 1, EPW), jnp.int32),
        pltpu.VMEM((CH, D), jnp.float32),
        pltpu.VMEM((CH, D), jnp.float32),
        pltpu.VMEM((CH, D), jnp.float32),
        pltpu.VMEM((CH, D), jnp.float32),
        pltpu.VMEM((CH, D), jnp.float32),
        pltpu.SemaphoreType.DMA,
        pltpu.SemaphoreType.DMA,
        pltpu.SemaphoreType.DMA,
        pltpu.SemaphoreType.DMA,
        pltpu.SemaphoreType.DMA,
    ],
)


_RB = 1000


def _mm0_body(x_ref, w_ref, out_ref):
    out_ref[...] = jnp.dot(x_ref[...], w_ref[...],
                           preferred_element_type=jnp.float32)


def _mm0_kernel(x, W):
    grid = (N // _RB,)
    return pl.pallas_call(
        _mm0_body,
        grid=grid,
        in_specs=[
            pl.BlockSpec((_RB, D), lambda i: (i, 0)),
            pl.BlockSpec((D, D), lambda i: (0, 0)),
        ],
        out_specs=pl.BlockSpec((_RB, D), lambda i: (i, 0)),
        out_shape=jax.ShapeDtypeStruct((N, D), jnp.float32),
    )(x, W)


def _scale_body(hist_ref, xw_ref, dis_ref, yw_ref):
    deg = 1.0 + jnp.sum(hist_ref[...], axis=0)
    dis = lax.rsqrt(deg)[:, None]
    dis_ref[...] = dis
    yw_ref[...] = dis * xw_ref[...]


def _scale_kernel(hist, xw):
    return pl.pallas_call(
        _scale_body,
        out_shape=[
            jax.ShapeDtypeStruct((N, 1), jnp.float32),
            jax.ShapeDtypeStruct((N, D), jnp.float32),
        ],
    )(hist, xw)


def _mid_body(p_ref, dis_ref, b_ref, w_ref, out_ref):
    dis = dis_ref[...]
    t = dis * (p_ref[0] + p_ref[1]) + b_ref[...]
    t = jnp.maximum(t, 0.0)
    tw = jnp.dot(t, w_ref[...], preferred_element_type=jnp.float32)
    out_ref[...] = dis * tw


def _mid_kernel(parts, dis, b, W):
    grid = (N // _RB,)
    return pl.pallas_call(
        _mid_body,
        grid=grid,
        in_specs=[
            pl.BlockSpec((NC, _RB, D), lambda i: (0, i, 0)),
            pl.BlockSpec((_RB, 1), lambda i: (i, 0)),
            pl.BlockSpec((1, D), lambda i: (0, 0)),
            pl.BlockSpec((D, D), lambda i: (0, 0)),
        ],
        out_specs=pl.BlockSpec((_RB, D), lambda i: (i, 0)),
        out_shape=jax.ShapeDtypeStruct((N, D), jnp.float32),
    )(parts, dis, b, W)


def _final_body(p_ref, dis_ref, b_ref, out_ref):
    out_ref[...] = dis_ref[...] * (p_ref[0] + p_ref[1]) + b_ref[...]


def _final_kernel(parts, dis, b):
    grid = (N // _RB,)
    return pl.pallas_call(
        _final_body,
        grid=grid,
        in_specs=[
            pl.BlockSpec((NC, _RB, D), lambda i: (0, i, 0)),
            pl.BlockSpec((_RB, 1), lambda i: (i, 0)),
            pl.BlockSpec((1, D), lambda i: (0, 0)),
        ],
        out_specs=pl.BlockSpec((_RB, D), lambda i: (i, 0)),
        out_shape=jax.ShapeDtypeStruct((N, D), jnp.float32),
    )(parts, dis, b)


def kernel(x, edge_index, W0, b0, W1, b1, W2, b2, W3, b3, W4, b4):
    edge_r = edge_index.reshape(2, NW, 1, EPW)
    zeros = jnp.zeros((N, D), jnp.float32)

    hist = _deg_kernel(edge_r)
    xw0 = _mm0_kernel(x, W0)
    dis, yw = _scale_kernel(hist, xw0)

    bs = [b0.reshape(1, D), b1.reshape(1, D), b2.reshape(1, D),
          b3.reshape(1, D), b4.reshape(1, D)]
    Ws = [W0, W1, W2, W3, W4]
    for l in range(4):
        parts = _gs_kernel(yw, edge_r, zeros)
        yw = _mid_kernel(parts, dis, bs[l], Ws[l + 1])
    parts = _gs_kernel(yw, edge_r, zeros)
    return _final_kernel(parts, dis, bs[4])


# speedup vs baseline: 1.1100x; 1.0419x over previous
import functools

import jax
import jax.numpy as jnp
from jax import lax
from jax.experimental import pallas as pl
from jax.experimental.pallas import tpu as pltpu
from jax.experimental.pallas import tpu_sc as plsc

N = 10000
E = 320000
D = 128
NC, NS = 2, 16
NW = NC * NS
EPW = E // NW
CH = 40
NCHUNK = EPW // CH
NBUF = 5
ROWS_PT = 624
TAIL0 = ROWS_PT * NS
TAILN = N - TAIL0

_sc_mesh = plsc.VectorSubcoreMesh(core_axis_name="c", subcore_axis_name="s")


def _deg_body(edge_hbm, hist_hbm, idx_v, hist_v, isem):
    c = lax.axis_index("c")
    s = lax.axis_index("s")
    wid = c * NS + s
    cp = pltpu.async_copy(edge_hbm.at[1, wid], idx_v, isem)

    zeros16 = jnp.zeros((16,), jnp.float32)

    @pl.loop(0, N // 16)
    def _zero(i):
        hist_v[pl.ds(i * 16, 16)] = zeros16

    cp.wait()

    ones16 = jnp.ones((16,), jnp.float32)

    @pl.loop(0, EPW // 16)
    def _acc(i):
        idx = idx_v[0, pl.ds(i * 16, 16)]
        plsc.addupdate_scatter(hist_v, [idx], ones16)

    pltpu.sync_copy(hist_v, hist_hbm.at[wid])


_deg_kernel = pl.kernel(
    _deg_body,
    out_type=jax.ShapeDtypeStruct((NW, N), jnp.float32),
    mesh=_sc_mesh,
    compiler_params=pltpu.CompilerParams(needs_layout_passes=False),
    scratch_types=[
        pltpu.VMEM((1, EPW), jnp.int32),
        pltpu.VMEM((N,), jnp.float32),
        pltpu.SemaphoreType.DMA,
    ],
)


def _gs_body(yw_hbm, edge_hbm, parts_hbm,
             acc, idx_v, buf0, buf1, buf2, buf3, buf4,
             gsem0, gsem1, gsem2, gsem3, gsem4):
    c = lax.axis_index("c")
    s = lax.axis_index("s")
    wid = c * NS + s
    src_v = idx_v.at[0]
    dst_v = idx_v.at[1]
    pltpu.sync_copy(edge_hbm.at[0, wid], src_v)

    bufs = (buf0, buf1, buf2, buf3, buf4)
    gsems = (gsem0, gsem1, gsem2, gsem3, gsem4)

    zero16 = jnp.zeros((16,), jnp.float32)

    @pl.loop(0, CH)
    def _z(r):
        for f in range(D // 16):
            buf0[r, pl.ds(f * 16, 16)] = zero16

    row0 = s * ROWS_PT
    NZ = ROWS_PT // CH
    REM = ROWS_PT - NZ * CH

    @pl.loop(0, NZ)
    def _zi(i):
        pltpu.sync_copy(buf0, acc.at[pl.ds(row0 + i * CH, CH)])

    pltpu.sync_copy(buf0.at[pl.ds(0, REM)],
                    acc.at[pl.ds(row0 + NZ * CH, REM)])

    @pl.when(s == NS - 1)
    def _():
        pltpu.sync_copy(buf0.at[pl.ds(0, TAILN)],
                        acc.at[pl.ds(TAIL0, TAILN)])

    for b in range(NBUF):
        pltpu.async_copy(yw_hbm.at[src_v.at[0, pl.ds(b * CH, CH)]],
                         bufs[b], gsems[b])

    pltpu.sync_copy(edge_hbm.at[1, wid], dst_v)

    plsc.subcore_barrier()

    @pl.loop(0, NCHUNK, step=NBUF)
    def _main(j):
        for b in range(NBUF):
            k = j + b
            pltpu.make_async_copy(
                yw_hbm.at[src_v.at[0, pl.ds(k * CH, CH)]],
                bufs[b], gsems[b]).wait()
            pltpu.sync_copy(bufs[b],
                            acc.at[dst_v.at[0, pl.ds(k * CH, CH)]],
                            add=True)

            @pl.when(k + NBUF < NCHUNK)
            def _():
                pltpu.async_copy(
                    yw_hbm.at[src_v.at[0, pl.ds((k + NBUF) * CH, CH)]],
                    bufs[b], gsems[b])

    plsc.subcore_barrier()
    pltpu.sync_copy(acc.at[pl.ds(row0, ROWS_PT)],
                    parts_hbm.at[c, pl.ds(row0, ROWS_PT)])

    @pl.when(s == NS - 1)
    def _():
        pltpu.sync_copy(acc.at[pl.ds(TAIL0, TAILN)],
                        parts_hbm.at[c, pl.ds(TAIL0, TAILN)])


_gs_kernel = pl.kernel(
    _gs_body,
    out_type=jax.ShapeDtypeStruct((NC, N, D), jnp.float32),
    mesh=_sc_mesh,
    compiler_params=pltpu.CompilerParams(use_tc_tiling_on_sc=False),
    scratch_types=[
        pltpu.VMEM_SHARED((N, D), jnp.float32),
        pltpu.VMEM((2, 1, EPW), jnp.int32),
        pltpu.VMEM((CH, D), jnp.float32),
        pltpu.VMEM((CH, D), jnp.float32),
        pltpu.VMEM((CH, D), jnp.float32),
        pltpu.VMEM((CH, D), jnp.float32),
        pltpu.VMEM((CH, D), jnp.float32),
        pltpu.SemaphoreType.DMA,
        pltpu.SemaphoreType.DMA,
        pltpu.SemaphoreType.DMA,
        pltpu.SemaphoreType.DMA,
        pltpu.SemaphoreType.DMA,
    ],
)


_RB = 2000


def _mm0_body(x_ref, w_ref, out_ref):
    out_ref[...] = jnp.dot(x_ref[...], w_ref[...],
                           preferred_element_type=jnp.float32)


def _mm0_kernel(x, W):
    grid = (N // _RB,)
    return pl.pallas_call(
        _mm0_body,
        grid=grid,
        in_specs=[
            pl.BlockSpec((_RB, D), lambda i: (i, 0)),
            pl.BlockSpec((D, D), lambda i: (0, 0)),
        ],
        out_specs=pl.BlockSpec((_RB, D), lambda i: (i, 0)),
        out_shape=jax.ShapeDtypeStruct((N, D), jnp.float32),
    )(x, W)


def _scale_body(hist_ref, xw_ref, dis_ref, yw_ref):
    deg = 1.0 + jnp.sum(hist_ref[...], axis=0)
    dis = lax.rsqrt(deg)[:, None]
    dis_ref[...] = dis
    yw_ref[...] = dis * xw_ref[...]


def _scale_kernel(hist, xw):
    return pl.pallas_call(
        _scale_body,
        out_shape=[
            jax.ShapeDtypeStruct((N, 1), jnp.float32),
            jax.ShapeDtypeStruct((N, D), jnp.float32),
        ],
    )(hist, xw)


def _mid_body(p_ref, yw_ref, dis_ref, b_ref, w_ref, out_ref):
    dis = dis_ref[...]
    t = dis * (p_ref[0] + p_ref[1] + yw_ref[...]) + b_ref[...]
    t = jnp.maximum(t, 0.0)
    tw = jnp.dot(t, w_ref[...], preferred_element_type=jnp.float32)
    out_ref[...] = dis * tw


def _mid_kernel(parts, yw, dis, b, W):
    grid = (N // _RB,)
    return pl.pallas_call(
        _mid_body,
        grid=grid,
        in_specs=[
            pl.BlockSpec((NC, _RB, D), lambda i: (0, i, 0)),
            pl.BlockSpec((_RB, D), lambda i: (i, 0)),
            pl.BlockSpec((_RB, 1), lambda i: (i, 0)),
            pl.BlockSpec((1, D), lambda i: (0, 0)),
            pl.BlockSpec((D, D), lambda i: (0, 0)),
        ],
        out_specs=pl.BlockSpec((_RB, D), lambda i: (i, 0)),
        out_shape=jax.ShapeDtypeStruct((N, D), jnp.float32),
    )(parts, yw, dis, b, W)


def _final_body(p_ref, yw_ref, dis_ref, b_ref, out_ref):
    out_ref[...] = (dis_ref[...] * (p_ref[0] + p_ref[1] + yw_ref[...])
                    + b_ref[...])


def _final_kernel(parts, yw, dis, b):
    grid = (N // _RB,)
    return pl.pallas_call(
        _final_body,
        grid=grid,
        in_specs=[
            pl.BlockSpec((NC, _RB, D), lambda i: (0, i, 0)),
            pl.BlockSpec((_RB, D), lambda i: (i, 0)),
            pl.BlockSpec((_RB, 1), lambda i: (i, 0)),
            pl.BlockSpec((1, D), lambda i: (0, 0)),
        ],
        out_specs=pl.BlockSpec((_RB, D), lambda i: (i, 0)),
        out_shape=jax.ShapeDtypeStruct((N, D), jnp.float32),
    )(parts, yw, dis, b)


def kernel(x, edge_index, W0, b0, W1, b1, W2, b2, W3, b3, W4, b4):
    edge_r = edge_index.reshape(2, NW, 1, EPW)

    hist = _deg_kernel(edge_r)
    xw0 = _mm0_kernel(x, W0)
    dis, yw = _scale_kernel(hist, xw0)

    bs = [b0.reshape(1, D), b1.reshape(1, D), b2.reshape(1, D),
          b3.reshape(1, D), b4.reshape(1, D)]
    Ws = [W0, W1, W2, W3, W4]
    for l in range(4):
        parts = _gs_kernel(yw, edge_r)
        yw = _mid_kernel(parts, yw, dis, bs[l], Ws[l + 1])
    parts = _gs_kernel(yw, edge_r)
    return _final_kernel(parts, yw, dis, bs[4])


# speedup vs baseline: 1.1287x; 1.0168x over previous
import functools

import jax
import jax.numpy as jnp
from jax import lax
from jax.experimental import pallas as pl
from jax.experimental.pallas import tpu as pltpu
from jax.experimental.pallas import tpu_sc as plsc

N = 10000
E = 320000
D = 128
NC, NS = 2, 16
NW = NC * NS
EPW = E // NW
CH = 40
NCHUNK = EPW // CH
NBUF = 5
ROWS_PT = 624
TAIL0 = ROWS_PT * NS
TAILN = N - TAIL0
ALIGN = 128
ALEN = EPW + ALIGN - 16

_sc_mesh = plsc.VectorSubcoreMesh(core_axis_name="c", subcore_axis_name="s")


def _deg_body(edge_hbm, hist_hbm, idx_v, hist_v, isem):
    c = lax.axis_index("c")
    s = lax.axis_index("s")
    wid = c * NS + s
    base = wid * EPW
    abase = (base // ALIGN) * ALIGN
    shift = base - abase
    cp = pltpu.async_copy(edge_hbm.at[:, pl.ds(abase, ALEN)], idx_v, isem)

    zeros16 = jnp.zeros((16,), jnp.float32)

    @pl.loop(0, N // 16)
    def _zero(i):
        hist_v[pl.ds(i * 16, 16)] = zeros16

    cp.wait()

    ones16 = jnp.ones((16,), jnp.float32)

    @pl.loop(0, EPW // 16)
    def _acc(i):
        idx = idx_v[1, pl.ds(shift + i * 16, 16)]
        plsc.addupdate_scatter(hist_v, [idx], ones16)

    pltpu.sync_copy(hist_v, hist_hbm.at[wid])


_deg_kernel = pl.kernel(
    _deg_body,
    out_type=jax.ShapeDtypeStruct((NW, N), jnp.float32),
    mesh=_sc_mesh,
    compiler_params=pltpu.CompilerParams(needs_layout_passes=False),
    scratch_types=[
        pltpu.VMEM((2, ALEN), jnp.int32),
        pltpu.VMEM((N,), jnp.float32),
        pltpu.SemaphoreType.DMA,
    ],
)


def _gs_body(yw_hbm, edge_hbm, parts_hbm,
             acc, idx_v, buf0, buf1, buf2, buf3, buf4,
             gsem0, gsem1, gsem2, gsem3, gsem4):
    c = lax.axis_index("c")
    s = lax.axis_index("s")
    wid = c * NS + s
    base = wid * EPW
    abase = (base // ALIGN) * ALIGN
    shift = base - abase
    pltpu.sync_copy(edge_hbm.at[:, pl.ds(abase, ALEN)], idx_v)
    src_v = idx_v.at[0]
    dst_v = idx_v.at[1]

    bufs = (buf0, buf1, buf2, buf3, buf4)
    gsems = (gsem0, gsem1, gsem2, gsem3, gsem4)

    zero16 = jnp.zeros((16,), jnp.float32)

    @pl.loop(0, CH)
    def _z(r):
        for f in range(D // 16):
            buf0[r, pl.ds(f * 16, 16)] = zero16

    row0 = s * ROWS_PT
    NZ = ROWS_PT // CH
    REM = ROWS_PT - NZ * CH

    @pl.loop(0, NZ)
    def _zi(i):
        pltpu.sync_copy(buf0, acc.at[pl.ds(row0 + i * CH, CH)])

    pltpu.sync_copy(buf0.at[pl.ds(0, REM)],
                    acc.at[pl.ds(row0 + NZ * CH, REM)])

    @pl.when(s == NS - 1)
    def _():
        pltpu.sync_copy(buf0.at[pl.ds(0, TAILN)],
                        acc.at[pl.ds(TAIL0, TAILN)])

    for b in range(NBUF):
        pltpu.async_copy(yw_hbm.at[src_v.at[pl.ds(shift + b * CH, CH)]],
                         bufs[b], gsems[b])

    plsc.subcore_barrier()

    @pl.loop(0, NCHUNK, step=NBUF)
    def _main(j):
        for b in range(NBUF):
            k = j + b
            pltpu.make_async_copy(
                yw_hbm.at[src_v.at[pl.ds(shift + k * CH, CH)]],
                bufs[b], gsems[b]).wait()
            pltpu.sync_copy(bufs[b],
                            acc.at[dst_v.at[pl.ds(shift + k * CH, CH)]],
                            add=True)

            @pl.when(k + NBUF < NCHUNK)
            def _():
                pltpu.async_copy(
                    yw_hbm.at[src_v.at[pl.ds(shift + (k + NBUF) * CH, CH)]],
                    bufs[b], gsems[b])

    plsc.subcore_barrier()
    pltpu.sync_copy(acc.at[pl.ds(row0, ROWS_PT)],
                    parts_hbm.at[c, pl.ds(row0, ROWS_PT)])

    @pl.when(s == NS - 1)
    def _():
        pltpu.sync_copy(acc.at[pl.ds(TAIL0, TAILN)],
                        parts_hbm.at[c, pl.ds(TAIL0, TAILN)])


_gs_kernel = pl.kernel(
    _gs_body,
    out_type=jax.ShapeDtypeStruct((NC, N, D), jnp.float32),
    mesh=_sc_mesh,
    compiler_params=pltpu.CompilerParams(use_tc_tiling_on_sc=False),
    scratch_types=[
        pltpu.VMEM_SHARED((N, D), jnp.float32),
        pltpu.VMEM((2, ALEN), jnp.int32),
        pltpu.VMEM((CH, D), jnp.float32),
        pltpu.VMEM((CH, D), jnp.float32),
        pltpu.VMEM((CH, D), jnp.float32),
        pltpu.VMEM((CH, D), jnp.float32),
        pltpu.VMEM((CH, D), jnp.float32),
        pltpu.SemaphoreType.DMA,
        pltpu.SemaphoreType.DMA,
        pltpu.SemaphoreType.DMA,
        pltpu.SemaphoreType.DMA,
        pltpu.SemaphoreType.DMA,
    ],
)


_RB = 2000


def _mm0_body(x_ref, w_ref, out_ref):
    out_ref[...] = jnp.dot(x_ref[...], w_ref[...],
                           preferred_element_type=jnp.float32)


def _mm0_kernel(x, W):
    grid = (N // _RB,)
    return pl.pallas_call(
        _mm0_body,
        grid=grid,
        in_specs=[
            pl.BlockSpec((_RB, D), lambda i: (i, 0)),
            pl.BlockSpec((D, D), lambda i: (0, 0)),
        ],
        out_specs=pl.BlockSpec((_RB, D), lambda i: (i, 0)),
        out_shape=jax.ShapeDtypeStruct((N, D), jnp.float32),
    )(x, W)


def _scale_body(hist_ref, xw_ref, dis_ref, yw_ref):
    deg = 1.0 + jnp.sum(hist_ref[...], axis=0)
    dis = lax.rsqrt(deg)[:, None]
    dis_ref[...] = dis
    yw_ref[...] = dis * xw_ref[...]


def _scale_kernel(hist, xw):
    return pl.pallas_call(
        _scale_body,
        out_shape=[
            jax.ShapeDtypeStruct((N, 1), jnp.float32),
            jax.ShapeDtypeStruct((N, D), jnp.float32),
        ],
    )(hist, xw)


def _mid_body(p_ref, yw_ref, dis_ref, b_ref, w_ref, out_ref):
    dis = dis_ref[...]
    t = dis * (p_ref[0] + p_ref[1] + yw_ref[...]) + b_ref[...]
    t = jnp.maximum(t, 0.0)
    tw = jnp.dot(t, w_ref[...], preferred_element_type=jnp.float32)
    out_ref[...] = dis * tw


def _mid_kernel(parts, yw, dis, b, W):
    grid = (N // _RB,)
    return pl.pallas_call(
        _mid_body,
        grid=grid,
        in_specs=[
            pl.BlockSpec((NC, _RB, D), lambda i: (0, i, 0)),
            pl.BlockSpec((_RB, D), lambda i: (i, 0)),
            pl.BlockSpec((_RB, 1), lambda i: (i, 0)),
            pl.BlockSpec((1, D), lambda i: (0, 0)),
            pl.BlockSpec((D, D), lambda i: (0, 0)),
        ],
        out_specs=pl.BlockSpec((_RB, D), lambda i: (i, 0)),
        out_shape=jax.ShapeDtypeStruct((N, D), jnp.float32),
    )(parts, yw, dis, b, W)


def _final_body(p_ref, yw_ref, dis_ref, b_ref, out_ref):
    out_ref[...] = (dis_ref[...] * (p_ref[0] + p_ref[1] + yw_ref[...])
                    + b_ref[...])


def _final_kernel(parts, yw, dis, b):
    grid = (N // _RB,)
    return pl.pallas_call(
        _final_body,
        grid=grid,
        in_specs=[
            pl.BlockSpec((NC, _RB, D), lambda i: (0, i, 0)),
            pl.BlockSpec((_RB, D), lambda i: (i, 0)),
            pl.BlockSpec((_RB, 1), lambda i: (i, 0)),
            pl.BlockSpec((1, D), lambda i: (0, 0)),
        ],
        out_specs=pl.BlockSpec((_RB, D), lambda i: (i, 0)),
        out_shape=jax.ShapeDtypeStruct((N, D), jnp.float32),
    )(parts, yw, dis, b)


def kernel(x, edge_index, W0, b0, W1, b1, W2, b2, W3, b3, W4, b4):
    hist = _deg_kernel(edge_index)
    xw0 = _mm0_kernel(x, W0)
    dis, yw = _scale_kernel(hist, xw0)

    bs = [b0.reshape(1, D), b1.reshape(1, D), b2.reshape(1, D),
          b3.reshape(1, D), b4.reshape(1, D)]
    Ws = [W0, W1, W2, W3, W4]
    for l in range(4):
        parts = _gs_kernel(yw, edge_index)
        yw = _mid_kernel(parts, yw, dis, bs[l], Ws[l + 1])
    parts = _gs_kernel(yw, edge_index)
    return _final_kernel(parts, yw, dis, bs[4])
